# baseline (speedup 1.0000x reference)
import functools

import jax
import jax.numpy as jnp
from jax import lax
from jax.experimental import pallas as pl
from jax.experimental.pallas import tpu as pltpu
from jax.experimental.pallas import tpu_sc as plsc

_BQ = 2048
_NC = 2
_NS = 16
_NW = _NC * _NS
_L = 16
_PIECE = 64


def _labels_block(x_ref, p_ref, lab_ref):
    x = x_ref[...]
    p = p_ref[...]
    x2 = jnp.sum(x * x, axis=1, keepdims=True)
    p2 = jnp.sum(p * p, axis=1)[None, :]
    dot = jax.lax.dot_general(
        x, p, (((1,), (1,)), ((), ())),
        preferred_element_type=jnp.float32)
    d2 = x2 + p2 - 2.0 * dot
    lab = jnp.argmin(d2, axis=1).astype(jnp.int32) + 1
    lab_ref[...] = lab[None, None, :]


def _onehot_write(n_out, rows_w, lab_hbm, out_hbm, buf, lab_v):
    wid = lax.axis_index("s") * _NC + lax.axis_index("c")
    base_row = wid * rows_w
    pelems = _PIECE * n_out
    pltpu.sync_copy(
        lab_hbm.at[pl.ds(pl.multiple_of(base_row, rows_w), rows_w)], lab_v)

    zeros = jnp.zeros((_L,), jnp.float32)
    ones = jnp.ones((_L,), jnp.float32)
    iota = lax.iota(jnp.int32, _L)

    def _zero(i, carry):
        buf[pl.ds(pl.multiple_of(i * _L, _L), _L)] = zeros
        return carry

    lax.fori_loop(0, pelems // _L, _zero, None)

    for piece in range(rows_w // _PIECE):
        locs = []
        for c in range(_PIECE // _L):
            lab16 = lab_v[pl.ds(piece * _PIECE + c * _L, _L)]
            loc16 = (iota + c * _L) * n_out + lab16
            locs.append(loc16)
            plsc.store_scatter(buf, [loc16], ones)
        off = (base_row + piece * _PIECE) * n_out
        pltpu.sync_copy(buf, out_hbm.at[pl.ds(pl.multiple_of(off, 8), pelems)])
        for loc16 in locs:
            plsc.store_scatter(buf, [loc16], zeros)


def kernel(x, protos):
    q, d = x.shape
    k, _ = protos.shape
    n_out = k + 1
    ni = q // _BQ
    labs = pl.pallas_call(
        _labels_block,
        grid=(ni,),
        in_specs=[
            pl.BlockSpec((_BQ, d), lambda i: (i, 0)),
            pl.BlockSpec((k, d), lambda i: (0, 0)),
        ],
        out_specs=pl.BlockSpec((1, 1, _BQ), lambda i: (i, 0, 0)),
        out_shape=jax.ShapeDtypeStruct((ni, 1, _BQ), jnp.int32),
        compiler_params=pltpu.CompilerParams(
            dimension_semantics=("parallel",)),
    )(x, protos).reshape(q)

    rows_w = q // _NW
    writer = pl.kernel(
        functools.partial(_onehot_write, n_out, rows_w),
        out_type=jax.ShapeDtypeStruct((q * n_out,), jnp.float32),
        mesh=plsc.VectorSubcoreMesh(core_axis_name="c", subcore_axis_name="s"),
        compiler_params=pltpu.CompilerParams(needs_layout_passes=False),
        scratch_types=[
            pltpu.VMEM((_PIECE * n_out,), jnp.float32),
            pltpu.VMEM((rows_w,), jnp.int32),
        ],
    )
    return writer(labs).reshape(q, n_out)
